# baseline (speedup 1.0000x reference)
import functools

import jax
import jax.numpy as jnp
from jax import lax
from jax.experimental import pallas as pl
from jax.experimental.pallas import tpu as pltpu
from jax.experimental.pallas import tpu_sc as plsc

B, L = 16384, 50
L_PAD = 56
COLOR_VOCAB = 1000
EDIM_S, EDIM_C, EDIM_C_PAD, HID, N_LAB = 32, 8, 16, 128, 128

NBUF = 8
NC, NS = 2, 16
NW = NC * NS
BAGS_PER_W = B // NW
IDS_PER_W = BAGS_PER_W * L_PAD
CH_BAGS = 2
CH_ROWS = CH_BAGS * L_PAD
N_CHUNKS = BAGS_PER_W // CH_BAGS
LANES = 16


def _bag_sum_sc(sid_flat, cid_flat, emb_s_tab, emb_c_pad):
    mesh = plsc.VectorSubcoreMesh(
        core_axis_name="c", subcore_axis_name="s", num_cores=NC,
        num_subcores=NS)

    @functools.partial(
        pl.kernel,
        compiler_params=pltpu.CompilerParams(use_tc_tiling_on_sc=False),
        out_type=(
            jax.ShapeDtypeStruct((B, EDIM_S), jnp.bfloat16),
            jax.ShapeDtypeStruct((B * 2, EDIM_C_PAD), jnp.bfloat16),
        ),
        mesh=mesh,
        scratch_types=(
            [pltpu.VMEM((IDS_PER_W,), jnp.int32)] * 2
            + [pltpu.VMEM((CH_ROWS,), jnp.int32)] * (2 * NBUF)
            + [pltpu.VMEM((CH_ROWS, EDIM_S), jnp.bfloat16)] * NBUF
            + [pltpu.VMEM((CH_ROWS, EDIM_C_PAD), jnp.bfloat16)] * NBUF
            + [pltpu.VMEM((BAGS_PER_W, EDIM_S), jnp.bfloat16),
               pltpu.VMEM((BAGS_PER_W * 2, EDIM_C_PAD), jnp.bfloat16)]
            + [pltpu.VMEM((COLOR_VOCAB, EDIM_C_PAD), jnp.bfloat16)]
            + [pltpu.VMEM_SHARED((COLOR_VOCAB, EDIM_C_PAD), jnp.bfloat16)]
            + [pltpu.SemaphoreType.DMA] * (1 + 2 * NBUF)
        ),
    )
    def bag_kernel(sid_hbm, cid_hbm, embs_hbm, embc_hbm, outs_hbm, outc_hbm,
                   *refs):
        sid_all, cid_all = refs[0], refs[1]
        gs = refs[2:2 + NBUF]
        gc = refs[2 + NBUF:2 + 2 * NBUF]
        rs = refs[2 + 2 * NBUF:2 + 3 * NBUF]
        rc = refs[2 + 3 * NBUF:2 + 4 * NBUF]
        stage_s, stage_c = refs[2 + 4 * NBUF], refs[3 + 4 * NBUF]
        ec_tmp, ec_sh = refs[4 + 4 * NBUF], refs[5 + 4 * NBUF]
        sem_in = refs[6 + 4 * NBUF]
        sgs = refs[7 + 4 * NBUF:7 + 5 * NBUF]
        sgc = refs[7 + 5 * NBUF:7 + 6 * NBUF]

        wid = lax.axis_index("s") * NC + lax.axis_index("c")
        id_base = wid * IDS_PER_W
        bag_base = wid * BAGS_PER_W

        @pl.when(lax.axis_index("s") == 0)
        def _():
            pltpu.async_copy(embc_hbm, ec_tmp, sem_in).wait()
            pltpu.sync_copy(ec_tmp, ec_sh)

        pltpu.async_copy(sid_hbm.at[pl.ds(id_base, IDS_PER_W)], sid_all,
                         sem_in).wait()
        pltpu.async_copy(cid_hbm.at[pl.ds(id_base, IDS_PER_W)], cid_all,
                         sem_in).wait()
        plsc.subcore_barrier()

        def prep_and_fire(i, p):
            off = i * CH_ROWS
            for j in range(CH_ROWS // LANES):
                s = sid_all[pl.ds(off + j * LANES, LANES)]
                c = cid_all[pl.ds(off + j * LANES, LANES)]
                gs[p][pl.ds(j * LANES, LANES)] = s
                gc[p][pl.ds(j * LANES, LANES)] = jnp.where(
                    s == 0, jnp.zeros_like(c), c)
            pltpu.async_copy(embs_hbm.at[gs[p]], rs[p], sgs[p])
            pltpu.async_copy(ec_sh.at[gc[p]], rc[p], sgc[p])

        def reduce_chunk(j, p):
            pltpu.make_async_copy(embs_hbm.at[gs[p]], rs[p], sgs[p]).wait()
            pltpu.make_async_copy(ec_sh.at[gc[p]], rc[p], sgc[p]).wait()
            for q in range(CH_BAGS):
                row0 = q * L_PAD
                acc = [rs[p][row0 + k, pl.ds(0, EDIM_S)] for k in range(4)]
                for k in range(4, L_PAD):
                    acc[k % 4] = acc[k % 4] + rs[p][row0 + k,
                                                    pl.ds(0, EDIM_S)]
                stage_s[j * CH_BAGS + q, pl.ds(0, EDIM_S)] = (
                    (acc[0] + acc[1]) + (acc[2] + acc[3]))
                accc = [rc[p][pl.ds(row0 + 2 * k, 2), pl.ds(0, LANES)]
                        for k in range(4)]
                for k in range(4, L_PAD // 2):
                    accc[k % 4] = accc[k % 4] + rc[p][
                        pl.ds(row0 + 2 * k, 2), pl.ds(0, LANES)]
                totc = (accc[0] + accc[1]) + (accc[2] + accc[3])
                stage_c[pl.ds((j * CH_BAGS + q) * 2, 2),
                        pl.ds(0, LANES)] = totc

        for k in range(NBUF - 1):
            prep_and_fire(k, k)

        def body(g2, _):
            base = g2 * NBUF
            for p in range(NBUF):
                i = base + p
                j = i + NBUF - 1

                @pl.when(i <= N_CHUNKS - NBUF)
                def _():
                    prep_and_fire(j, (p + NBUF - 1) % NBUF)

                reduce_chunk(i, p)
            return _

        lax.fori_loop(0, N_CHUNKS // NBUF, body, None)

        pltpu.async_copy(stage_s, outs_hbm.at[pl.ds(bag_base, BAGS_PER_W)],
                         sem_in).wait()
        pltpu.async_copy(
            stage_c, outc_hbm.at[pl.ds(bag_base * 2, BAGS_PER_W * 2)],
            sem_in).wait()

    return bag_kernel(sid_flat, cid_flat, emb_s_tab, emb_c_pad)


def _mlp_body(ss_ref, sc_ref, len_ref, w1s_ref, w1c_ref, b1_ref, w2_ref,
              b2_ref, out_ref):
    inv = 1.0 / len_ref[...]
    ms = ss_ref[...].astype(jnp.float32) * inv
    scf = sc_ref[...].astype(jnp.float32)
    mc = (scf[:, :EDIM_C_PAD] + scf[:, EDIM_C_PAD:]) * inv
    h = (jnp.dot(ms, w1s_ref[...], preferred_element_type=jnp.float32)
         + jnp.dot(mc, w1c_ref[...], preferred_element_type=jnp.float32)
         + b1_ref[...])
    h = jnp.maximum(h, 0.0)
    out_ref[...] = (jnp.dot(h, w2_ref[...],
                            preferred_element_type=jnp.float32) + b2_ref[...])


def _mlp_tc(sum_s, sum_c, lens_f, W1s, W1c, b1, W2, b2):
    BM = 2048
    grid = (B // BM,)
    return pl.pallas_call(
        _mlp_body,
        grid=grid,
        in_specs=[
            pl.BlockSpec((BM, EDIM_S), lambda i: (i, 0)),
            pl.BlockSpec((BM, 2 * EDIM_C_PAD), lambda i: (i, 0)),
            pl.BlockSpec((BM, 1), lambda i: (i, 0)),
            pl.BlockSpec((EDIM_S, HID), lambda i: (0, 0)),
            pl.BlockSpec((EDIM_C_PAD, HID), lambda i: (0, 0)),
            pl.BlockSpec((1, HID), lambda i: (0, 0)),
            pl.BlockSpec((HID, N_LAB), lambda i: (0, 0)),
            pl.BlockSpec((1, N_LAB), lambda i: (0, 0)),
        ],
        out_specs=pl.BlockSpec((BM, N_LAB), lambda i: (i, 0)),
        out_shape=jax.ShapeDtypeStruct((B, N_LAB), jnp.float32),
    )(sum_s, sum_c, lens_f, W1s, W1c, b1, W2, b2)


def kernel(shape_ids, color_ids, lens, emb_s, emb_c, W1, b1, W2, b2):
    sid_flat = jnp.pad(shape_ids, ((0, 0), (0, L_PAD - L))).reshape(-1)
    cid_flat = jnp.pad(color_ids, ((0, 0), (0, L_PAD - L))).reshape(-1)
    emb_c_pad = jnp.pad(emb_c, ((0, 0), (0, EDIM_C_PAD - EDIM_C))).astype(
        jnp.bfloat16)
    sum_s, sum_c = _bag_sum_sc(sid_flat, cid_flat,
                               emb_s.astype(jnp.bfloat16), emb_c_pad)
    sum_c = sum_c.reshape(B, 2 * EDIM_C_PAD)
    lens_f = lens.astype(jnp.float32).reshape(-1, 1)
    W1s = W1[:EDIM_S]
    W1c = jnp.pad(W1[EDIM_S:], ((0, EDIM_C_PAD - EDIM_C), (0, 0)))
    return _mlp_tc(sum_s, sum_c, lens_f, W1s, W1c, b1.reshape(1, -1), W2,
                   b2.reshape(1, -1))


# speedup vs baseline: 1.0024x; 1.0024x over previous
import functools

import jax
import jax.numpy as jnp
from jax import lax
from jax.experimental import pallas as pl
from jax.experimental.pallas import tpu as pltpu
from jax.experimental.pallas import tpu_sc as plsc

B, L = 16384, 50
L_PAD = 56
COLOR_VOCAB = 1000
EDIM_S, EDIM_C, EDIM_C_PAD, HID, N_LAB = 32, 8, 16, 128, 128

NBUF = 4
SPLITS = ((0, 32), (32, 32), (64, 32), (96, 16))
NC, NS = 2, 16
NW = NC * NS
BAGS_PER_W = B // NW
IDS_PER_W = BAGS_PER_W * L_PAD
CH_BAGS = 2
CH_ROWS = CH_BAGS * L_PAD
N_CHUNKS = BAGS_PER_W // CH_BAGS
LANES = 16


def _bag_sum_sc(sid_flat, cid_flat, emb_s_tab, emb_c_pad):
    mesh = plsc.VectorSubcoreMesh(
        core_axis_name="c", subcore_axis_name="s", num_cores=NC,
        num_subcores=NS)

    @functools.partial(
        pl.kernel,
        compiler_params=pltpu.CompilerParams(use_tc_tiling_on_sc=False),
        out_type=(
            jax.ShapeDtypeStruct((B, EDIM_S), jnp.bfloat16),
            jax.ShapeDtypeStruct((B * 2, EDIM_C_PAD), jnp.bfloat16),
        ),
        mesh=mesh,
        scratch_types=(
            [pltpu.VMEM((IDS_PER_W,), jnp.int32)] * 2
            + [pltpu.VMEM((CH_ROWS,), jnp.int32)] * (2 * NBUF)
            + [pltpu.VMEM((CH_ROWS, EDIM_S), jnp.bfloat16)] * NBUF
            + [pltpu.VMEM((CH_ROWS, EDIM_C_PAD), jnp.bfloat16)] * NBUF
            + [pltpu.VMEM((BAGS_PER_W, EDIM_S), jnp.bfloat16),
               pltpu.VMEM((BAGS_PER_W * 2, EDIM_C_PAD), jnp.bfloat16)]
            + [pltpu.VMEM((COLOR_VOCAB, EDIM_C_PAD), jnp.bfloat16)]
            + [pltpu.VMEM_SHARED((COLOR_VOCAB, EDIM_C_PAD), jnp.bfloat16)]
            + [pltpu.SemaphoreType.DMA] * (1 + 2 * NBUF)
        ),
    )
    def bag_kernel(sid_hbm, cid_hbm, embs_hbm, embc_hbm, outs_hbm, outc_hbm,
                   *refs):
        sid_all, cid_all = refs[0], refs[1]
        gs = refs[2:2 + NBUF]
        gc = refs[2 + NBUF:2 + 2 * NBUF]
        rs = refs[2 + 2 * NBUF:2 + 3 * NBUF]
        rc = refs[2 + 3 * NBUF:2 + 4 * NBUF]
        stage_s, stage_c = refs[2 + 4 * NBUF], refs[3 + 4 * NBUF]
        ec_tmp, ec_sh = refs[4 + 4 * NBUF], refs[5 + 4 * NBUF]
        sem_in = refs[6 + 4 * NBUF]
        sgs = refs[7 + 4 * NBUF:7 + 5 * NBUF]
        sgc = refs[7 + 5 * NBUF:7 + 6 * NBUF]

        wid = lax.axis_index("s") * NC + lax.axis_index("c")
        id_base = wid * IDS_PER_W
        bag_base = wid * BAGS_PER_W

        @pl.when(lax.axis_index("s") == 0)
        def _():
            pltpu.async_copy(embc_hbm, ec_tmp, sem_in).wait()
            pltpu.sync_copy(ec_tmp, ec_sh)

        pltpu.async_copy(sid_hbm.at[pl.ds(id_base, IDS_PER_W)], sid_all,
                         sem_in).wait()
        pltpu.async_copy(cid_hbm.at[pl.ds(id_base, IDS_PER_W)], cid_all,
                         sem_in).wait()
        plsc.subcore_barrier()

        def prep_and_fire(i, p):
            off = i * CH_ROWS
            for j in range(CH_ROWS // LANES):
                s = sid_all[pl.ds(off + j * LANES, LANES)]
                c = cid_all[pl.ds(off + j * LANES, LANES)]
                gs[p][pl.ds(j * LANES, LANES)] = s
                gc[p][pl.ds(j * LANES, LANES)] = jnp.where(
                    s == 0, jnp.zeros_like(c), c)
            for off, n in SPLITS:
                sl = pl.ds(off, n)
                pltpu.async_copy(embs_hbm.at[gs[p].at[sl]], rs[p].at[sl],
                                 sgs[p])
                pltpu.async_copy(ec_sh.at[gc[p].at[sl]], rc[p].at[sl],
                                 sgc[p])

        def reduce_chunk(j, p):
            for off, n in SPLITS:
                sl = pl.ds(off, n)
                pltpu.make_async_copy(embs_hbm.at[gs[p].at[sl]],
                                      rs[p].at[sl], sgs[p]).wait()
                pltpu.make_async_copy(ec_sh.at[gc[p].at[sl]],
                                      rc[p].at[sl], sgc[p]).wait()
            for q in range(CH_BAGS):
                row0 = q * L_PAD
                acc = [rs[p][row0 + k, pl.ds(0, EDIM_S)] for k in range(4)]
                for k in range(4, L_PAD):
                    acc[k % 4] = acc[k % 4] + rs[p][row0 + k,
                                                    pl.ds(0, EDIM_S)]
                stage_s[j * CH_BAGS + q, pl.ds(0, EDIM_S)] = (
                    (acc[0] + acc[1]) + (acc[2] + acc[3]))
                accc = [rc[p][pl.ds(row0 + 2 * k, 2), pl.ds(0, LANES)]
                        for k in range(4)]
                for k in range(4, L_PAD // 2):
                    accc[k % 4] = accc[k % 4] + rc[p][
                        pl.ds(row0 + 2 * k, 2), pl.ds(0, LANES)]
                totc = (accc[0] + accc[1]) + (accc[2] + accc[3])
                stage_c[pl.ds((j * CH_BAGS + q) * 2, 2),
                        pl.ds(0, LANES)] = totc

        for k in range(NBUF - 1):
            prep_and_fire(k, k)

        def body(g2, _):
            base = g2 * NBUF
            for p in range(NBUF):
                i = base + p
                j = i + NBUF - 1

                @pl.when(i <= N_CHUNKS - NBUF)
                def _():
                    prep_and_fire(j, (p + NBUF - 1) % NBUF)

                reduce_chunk(i, p)
            return _

        lax.fori_loop(0, N_CHUNKS // NBUF, body, None)

        pltpu.async_copy(stage_s, outs_hbm.at[pl.ds(bag_base, BAGS_PER_W)],
                         sem_in).wait()
        pltpu.async_copy(
            stage_c, outc_hbm.at[pl.ds(bag_base * 2, BAGS_PER_W * 2)],
            sem_in).wait()

    return bag_kernel(sid_flat, cid_flat, emb_s_tab, emb_c_pad)


def _mlp_body(ss_ref, sc_ref, len_ref, w1s_ref, w1c_ref, b1_ref, w2_ref,
              b2_ref, out_ref):
    inv = 1.0 / len_ref[...]
    ms = ss_ref[...].astype(jnp.float32) * inv
    scf = sc_ref[...].astype(jnp.float32)
    mc = (scf[:, :EDIM_C_PAD] + scf[:, EDIM_C_PAD:]) * inv
    h = (jnp.dot(ms, w1s_ref[...], preferred_element_type=jnp.float32)
         + jnp.dot(mc, w1c_ref[...], preferred_element_type=jnp.float32)
         + b1_ref[...])
    h = jnp.maximum(h, 0.0)
    out_ref[...] = (jnp.dot(h, w2_ref[...],
                            preferred_element_type=jnp.float32) + b2_ref[...])


def _mlp_tc(sum_s, sum_c, lens_f, W1s, W1c, b1, W2, b2):
    BM = 2048
    grid = (B // BM,)
    return pl.pallas_call(
        _mlp_body,
        grid=grid,
        in_specs=[
            pl.BlockSpec((BM, EDIM_S), lambda i: (i, 0)),
            pl.BlockSpec((BM, 2 * EDIM_C_PAD), lambda i: (i, 0)),
            pl.BlockSpec((BM, 1), lambda i: (i, 0)),
            pl.BlockSpec((EDIM_S, HID), lambda i: (0, 0)),
            pl.BlockSpec((EDIM_C_PAD, HID), lambda i: (0, 0)),
            pl.BlockSpec((1, HID), lambda i: (0, 0)),
            pl.BlockSpec((HID, N_LAB), lambda i: (0, 0)),
            pl.BlockSpec((1, N_LAB), lambda i: (0, 0)),
        ],
        out_specs=pl.BlockSpec((BM, N_LAB), lambda i: (i, 0)),
        out_shape=jax.ShapeDtypeStruct((B, N_LAB), jnp.float32),
    )(sum_s, sum_c, lens_f, W1s, W1c, b1, W2, b2)


def kernel(shape_ids, color_ids, lens, emb_s, emb_c, W1, b1, W2, b2):
    sid_flat = jnp.pad(shape_ids, ((0, 0), (0, L_PAD - L))).reshape(-1)
    cid_flat = jnp.pad(color_ids, ((0, 0), (0, L_PAD - L))).reshape(-1)
    emb_c_pad = jnp.pad(emb_c, ((0, 0), (0, EDIM_C_PAD - EDIM_C))).astype(
        jnp.bfloat16)
    sum_s, sum_c = _bag_sum_sc(sid_flat, cid_flat,
                               emb_s.astype(jnp.bfloat16), emb_c_pad)
    sum_c = sum_c.reshape(B, 2 * EDIM_C_PAD)
    lens_f = lens.astype(jnp.float32).reshape(-1, 1)
    W1s = W1[:EDIM_S]
    W1c = jnp.pad(W1[EDIM_S:], ((0, EDIM_C_PAD - EDIM_C), (0, 0)))
    return _mlp_tc(sum_s, sum_c, lens_f, W1s, W1c, b1.reshape(1, -1), W2,
                   b2.reshape(1, -1))


# speedup vs baseline: 1.0033x; 1.0009x over previous
import functools

import jax
import jax.numpy as jnp
from jax import lax
from jax.experimental import pallas as pl
from jax.experimental.pallas import tpu as pltpu
from jax.experimental.pallas import tpu_sc as plsc

B, L = 16384, 50
L_PAD = 56
COLOR_VOCAB = 1000
EDIM_S, EDIM_C, EDIM_C_PAD, HID, N_LAB = 32, 8, 16, 128, 128

NBUF = 4
SPLITS = tuple((16 * i, 16) for i in range(7))
NC, NS = 2, 16
NW = NC * NS
BAGS_PER_W = B // NW
IDS_PER_W = BAGS_PER_W * L_PAD
CH_BAGS = 2
CH_ROWS = CH_BAGS * L_PAD
N_CHUNKS = BAGS_PER_W // CH_BAGS
LANES = 16


def _bag_sum_sc(sid_flat, cid_flat, emb_s_tab, emb_c_pad):
    mesh = plsc.VectorSubcoreMesh(
        core_axis_name="c", subcore_axis_name="s", num_cores=NC,
        num_subcores=NS)

    @functools.partial(
        pl.kernel,
        compiler_params=pltpu.CompilerParams(use_tc_tiling_on_sc=False),
        out_type=(
            jax.ShapeDtypeStruct((B, EDIM_S), jnp.bfloat16),
            jax.ShapeDtypeStruct((B * 2, EDIM_C_PAD), jnp.bfloat16),
        ),
        mesh=mesh,
        scratch_types=(
            [pltpu.VMEM((IDS_PER_W,), jnp.int32)] * 2
            + [pltpu.VMEM((CH_ROWS,), jnp.int32)] * (2 * NBUF)
            + [pltpu.VMEM((CH_ROWS, EDIM_S), jnp.bfloat16)] * NBUF
            + [pltpu.VMEM((CH_ROWS, EDIM_C_PAD), jnp.bfloat16)] * NBUF
            + [pltpu.VMEM((BAGS_PER_W, EDIM_S), jnp.bfloat16),
               pltpu.VMEM((BAGS_PER_W * 2, EDIM_C_PAD), jnp.bfloat16)]
            + [pltpu.VMEM((COLOR_VOCAB, EDIM_C_PAD), jnp.bfloat16)]
            + [pltpu.VMEM_SHARED((COLOR_VOCAB, EDIM_C_PAD), jnp.bfloat16)]
            + [pltpu.SemaphoreType.DMA] * (1 + 2 * NBUF)
        ),
    )
    def bag_kernel(sid_hbm, cid_hbm, embs_hbm, embc_hbm, outs_hbm, outc_hbm,
                   *refs):
        sid_all, cid_all = refs[0], refs[1]
        gs = refs[2:2 + NBUF]
        gc = refs[2 + NBUF:2 + 2 * NBUF]
        rs = refs[2 + 2 * NBUF:2 + 3 * NBUF]
        rc = refs[2 + 3 * NBUF:2 + 4 * NBUF]
        stage_s, stage_c = refs[2 + 4 * NBUF], refs[3 + 4 * NBUF]
        ec_tmp, ec_sh = refs[4 + 4 * NBUF], refs[5 + 4 * NBUF]
        sem_in = refs[6 + 4 * NBUF]
        sgs = refs[7 + 4 * NBUF:7 + 5 * NBUF]
        sgc = refs[7 + 5 * NBUF:7 + 6 * NBUF]

        wid = lax.axis_index("s") * NC + lax.axis_index("c")
        id_base = wid * IDS_PER_W
        bag_base = wid * BAGS_PER_W

        @pl.when(lax.axis_index("s") == 0)
        def _():
            pltpu.async_copy(embc_hbm, ec_tmp, sem_in).wait()
            pltpu.sync_copy(ec_tmp, ec_sh)

        pltpu.async_copy(sid_hbm.at[pl.ds(id_base, IDS_PER_W)], sid_all,
                         sem_in).wait()
        pltpu.async_copy(cid_hbm.at[pl.ds(id_base, IDS_PER_W)], cid_all,
                         sem_in).wait()
        plsc.subcore_barrier()

        def prep_and_fire(i, p):
            off = i * CH_ROWS
            for j in range(CH_ROWS // LANES):
                s = sid_all[pl.ds(off + j * LANES, LANES)]
                c = cid_all[pl.ds(off + j * LANES, LANES)]
                gs[p][pl.ds(j * LANES, LANES)] = s
                gc[p][pl.ds(j * LANES, LANES)] = jnp.where(
                    s == 0, jnp.zeros_like(c), c)
            for off, n in SPLITS:
                sl = pl.ds(off, n)
                pltpu.async_copy(embs_hbm.at[gs[p].at[sl]], rs[p].at[sl],
                                 sgs[p])
                pltpu.async_copy(ec_sh.at[gc[p].at[sl]], rc[p].at[sl],
                                 sgc[p])

        def reduce_chunk(j, p):
            for off, n in SPLITS:
                sl = pl.ds(off, n)
                pltpu.make_async_copy(embs_hbm.at[gs[p].at[sl]],
                                      rs[p].at[sl], sgs[p]).wait()
                pltpu.make_async_copy(ec_sh.at[gc[p].at[sl]],
                                      rc[p].at[sl], sgc[p]).wait()
            for q in range(CH_BAGS):
                row0 = q * L_PAD
                acc = [rs[p][row0 + k, pl.ds(0, EDIM_S)] for k in range(4)]
                for k in range(4, L_PAD):
                    acc[k % 4] = acc[k % 4] + rs[p][row0 + k,
                                                    pl.ds(0, EDIM_S)]
                stage_s[j * CH_BAGS + q, pl.ds(0, EDIM_S)] = (
                    (acc[0] + acc[1]) + (acc[2] + acc[3]))
                accc = [rc[p][pl.ds(row0 + 2 * k, 2), pl.ds(0, LANES)]
                        for k in range(4)]
                for k in range(4, L_PAD // 2):
                    accc[k % 4] = accc[k % 4] + rc[p][
                        pl.ds(row0 + 2 * k, 2), pl.ds(0, LANES)]
                totc = (accc[0] + accc[1]) + (accc[2] + accc[3])
                stage_c[pl.ds((j * CH_BAGS + q) * 2, 2),
                        pl.ds(0, LANES)] = totc

        for k in range(NBUF - 1):
            prep_and_fire(k, k)

        def body(g2, _):
            base = g2 * NBUF
            for p in range(NBUF):
                i = base + p
                j = i + NBUF - 1

                @pl.when(i <= N_CHUNKS - NBUF)
                def _():
                    prep_and_fire(j, (p + NBUF - 1) % NBUF)

                reduce_chunk(i, p)
            return _

        lax.fori_loop(0, N_CHUNKS // NBUF, body, None)

        pltpu.async_copy(stage_s, outs_hbm.at[pl.ds(bag_base, BAGS_PER_W)],
                         sem_in).wait()
        pltpu.async_copy(
            stage_c, outc_hbm.at[pl.ds(bag_base * 2, BAGS_PER_W * 2)],
            sem_in).wait()

    return bag_kernel(sid_flat, cid_flat, emb_s_tab, emb_c_pad)


def _mlp_body(ss_ref, sc_ref, len_ref, w1s_ref, w1c_ref, b1_ref, w2_ref,
              b2_ref, out_ref):
    inv = 1.0 / len_ref[...]
    ms = ss_ref[...].astype(jnp.float32) * inv
    scf = sc_ref[...].astype(jnp.float32)
    mc = (scf[:, :EDIM_C_PAD] + scf[:, EDIM_C_PAD:]) * inv
    h = (jnp.dot(ms, w1s_ref[...], preferred_element_type=jnp.float32)
         + jnp.dot(mc, w1c_ref[...], preferred_element_type=jnp.float32)
         + b1_ref[...])
    h = jnp.maximum(h, 0.0)
    out_ref[...] = (jnp.dot(h, w2_ref[...],
                            preferred_element_type=jnp.float32) + b2_ref[...])


def _mlp_tc(sum_s, sum_c, lens_f, W1s, W1c, b1, W2, b2):
    BM = 2048
    grid = (B // BM,)
    return pl.pallas_call(
        _mlp_body,
        grid=grid,
        in_specs=[
            pl.BlockSpec((BM, EDIM_S), lambda i: (i, 0)),
            pl.BlockSpec((BM, 2 * EDIM_C_PAD), lambda i: (i, 0)),
            pl.BlockSpec((BM, 1), lambda i: (i, 0)),
            pl.BlockSpec((EDIM_S, HID), lambda i: (0, 0)),
            pl.BlockSpec((EDIM_C_PAD, HID), lambda i: (0, 0)),
            pl.BlockSpec((1, HID), lambda i: (0, 0)),
            pl.BlockSpec((HID, N_LAB), lambda i: (0, 0)),
            pl.BlockSpec((1, N_LAB), lambda i: (0, 0)),
        ],
        out_specs=pl.BlockSpec((BM, N_LAB), lambda i: (i, 0)),
        out_shape=jax.ShapeDtypeStruct((B, N_LAB), jnp.float32),
    )(sum_s, sum_c, lens_f, W1s, W1c, b1, W2, b2)


def kernel(shape_ids, color_ids, lens, emb_s, emb_c, W1, b1, W2, b2):
    sid_flat = jnp.pad(shape_ids, ((0, 0), (0, L_PAD - L))).reshape(-1)
    cid_flat = jnp.pad(color_ids, ((0, 0), (0, L_PAD - L))).reshape(-1)
    emb_c_pad = jnp.pad(emb_c, ((0, 0), (0, EDIM_C_PAD - EDIM_C))).astype(
        jnp.bfloat16)
    sum_s, sum_c = _bag_sum_sc(sid_flat, cid_flat,
                               emb_s.astype(jnp.bfloat16), emb_c_pad)
    sum_c = sum_c.reshape(B, 2 * EDIM_C_PAD)
    lens_f = lens.astype(jnp.float32).reshape(-1, 1)
    W1s = W1[:EDIM_S]
    W1c = jnp.pad(W1[EDIM_S:], ((0, EDIM_C_PAD - EDIM_C), (0, 0)))
    return _mlp_tc(sum_s, sum_c, lens_f, W1s, W1c, b1.reshape(1, -1), W2,
                   b2.reshape(1, -1))
